# baseline (device time: 34897 ns/iter reference)
import jax
import jax.numpy as jnp
from jax import lax
from jax.experimental import pallas as pl
from jax.experimental.pallas import tpu as pltpu

NC = 4
Y_ORDER = (0, 3, 1, 2)
XR = (0, 1)
ZR = (2, 3)


def kernel(x):
    m, n = x.shape
    n_out = n // 2
    qm = m // 4
    ck = qm // NC

    def body(x_ref, out_ref, local_sem,
             ysend, yrecv, xqs, xqr, zqs, zqr, xrs, xrr, zrs, zrr):
        my_x = lax.axis_index("x")
        my_y = lax.axis_index("y")
        my_z = lax.axis_index("z")
        ypeer = (my_x, 1 - my_y, my_z)
        xsib = (1 - my_x, my_y, my_z)
        zsib = (my_x, my_y, 1 - my_z)

        qoff = my_x * (2 * qm) + my_z * qm
        recvbase = (1 - my_y) * m
        row_q = recvbase + qoff
        row_xq = recvbase + (1 - my_x) * (2 * qm) + my_z * qm
        row_zq = recvbase + my_x * (2 * qm) + (1 - my_z) * qm

        local_copy = pltpu.make_async_copy(
            x_ref.at[:, pl.ds(my_y * n_out, n_out)],
            out_ref.at[pl.ds(my_y * m, m), :],
            local_sem,
        )
        local_copy.start()

        barrier_sem = pltpu.get_barrier_semaphore()
        for p in (ypeer, xsib, zsib):
            pl.semaphore_signal(
                barrier_sem, inc=1,
                device_id=p, device_id_type=pl.DeviceIdType.MESH,
            )
        pl.semaphore_wait(barrier_sem, 3)

        def rdma(src_row, dst_row, nrows, ssem, rsem, peer):
            return pltpu.make_async_remote_copy(
                src_ref=out_ref.at[pl.ds(src_row, nrows), :],
                dst_ref=out_ref.at[pl.ds(dst_row, nrows), :],
                send_sem=ssem,
                recv_sem=rsem,
                device_id=peer,
                device_id_type=pl.DeviceIdType.MESH,
            )

        y_rd = {}
        for c in Y_ORDER:
            r = pltpu.make_async_remote_copy(
                src_ref=x_ref.at[
                    pl.ds(qoff + c * ck, ck),
                    pl.ds((1 - my_y) * n_out, n_out),
                ],
                dst_ref=out_ref.at[pl.ds(my_y * m + qoff + c * ck, ck), :],
                send_sem=ysend.at[c],
                recv_sem=yrecv.at[c],
                device_id=ypeer,
                device_id_type=pl.DeviceIdType.MESH,
            )
            r.start()
            y_rd[c] = r

        xq_rd, zq_rd = {}, {}
        for c in Y_ORDER:
            y_rd[c].wait_recv()
            r = rdma(row_q + c * ck, row_q + c * ck, ck,
                     xqs.at[c], xqr.at[c], xsib)
            r.start()
            xq_rd[c] = r
            r = rdma(row_q + c * ck, row_q + c * ck, ck,
                     zqs.at[c], zqr.at[c], zsib)
            r.start()
            zq_rd[c] = r

        xr_rd, zr_rd = {}, {}
        zq_rd[0].wait_recv()
        xr_rd[0] = rdma(row_zq + 0 * ck, row_zq + 0 * ck, ck,
                        xrs.at[0], xrr.at[0], xsib)
        xr_rd[0].start()
        xq_rd[3].wait_recv()
        zr_rd[3] = rdma(row_xq + 3 * ck, row_xq + 3 * ck, ck,
                        zrs.at[3], zrr.at[3], zsib)
        zr_rd[3].start()
        zq_rd[1].wait_recv()
        xr_rd[1] = rdma(row_zq + 1 * ck, row_zq + 1 * ck, ck,
                        xrs.at[1], xrr.at[1], xsib)
        xr_rd[1].start()
        xq_rd[2].wait_recv()
        zr_rd[2] = rdma(row_xq + 2 * ck, row_xq + 2 * ck, ck,
                        zrs.at[2], zrr.at[2], zsib)
        zr_rd[2].start()

        for c in (0, 1):
            xq_rd[c].wait_recv()
        for c in (2, 3):
            zq_rd[c].wait_recv()
        for k in XR:
            xr_rd[k].wait_recv()
        for k in ZR:
            zr_rd[k].wait_recv()
        for c in range(NC):
            y_rd[c].wait_send()
            xq_rd[c].wait_send()
            zq_rd[c].wait_send()
        for k in XR:
            xr_rd[k].wait_send()
        for k in ZR:
            zr_rd[k].wait_send()
        local_copy.wait()

    return pl.pallas_call(
        body,
        out_shape=jax.ShapeDtypeStruct((2 * m, n_out), x.dtype),
        in_specs=[pl.BlockSpec(memory_space=pl.ANY)],
        out_specs=pl.BlockSpec(memory_space=pl.ANY),
        scratch_shapes=[
            pltpu.SemaphoreType.DMA,
            pltpu.SemaphoreType.DMA((NC,)),
            pltpu.SemaphoreType.DMA((NC,)),
            pltpu.SemaphoreType.DMA((NC,)),
            pltpu.SemaphoreType.DMA((NC,)),
            pltpu.SemaphoreType.DMA((NC,)),
            pltpu.SemaphoreType.DMA((NC,)),
            pltpu.SemaphoreType.DMA((NC,)),
            pltpu.SemaphoreType.DMA((NC,)),
            pltpu.SemaphoreType.DMA((NC,)),
            pltpu.SemaphoreType.DMA((NC,)),
        ],
        compiler_params=pltpu.CompilerParams(collective_id=0),
    )(x)


# device time: 33866 ns/iter; 1.0304x vs baseline; 1.0304x over previous
import jax
import jax.numpy as jnp
from jax import lax
from jax.experimental import pallas as pl
from jax.experimental.pallas import tpu as pltpu

NC = 8
Y_ORDER = tuple(
    c for i in range(NC // 2) for c in (i, NC - 1 - i)
)
XR = tuple(range(NC // 2))
ZR = tuple(range(NC // 2, NC))


def kernel(x):
    m, n = x.shape
    n_out = n // 2
    qm = m // 4
    ck = qm // NC

    def body(x_ref, out_ref, local_sem,
             ysend, yrecv, xqs, xqr, zqs, zqr, xrs, xrr, zrs, zrr):
        my_x = lax.axis_index("x")
        my_y = lax.axis_index("y")
        my_z = lax.axis_index("z")
        ypeer = (my_x, 1 - my_y, my_z)
        xsib = (1 - my_x, my_y, my_z)
        zsib = (my_x, my_y, 1 - my_z)

        qoff = my_x * (2 * qm) + my_z * qm
        recvbase = (1 - my_y) * m
        row_q = recvbase + qoff
        row_xq = recvbase + (1 - my_x) * (2 * qm) + my_z * qm
        row_zq = recvbase + my_x * (2 * qm) + (1 - my_z) * qm

        local_copy = pltpu.make_async_copy(
            x_ref.at[:, pl.ds(my_y * n_out, n_out)],
            out_ref.at[pl.ds(my_y * m, m), :],
            local_sem,
        )
        local_copy.start()

        barrier_sem = pltpu.get_barrier_semaphore()
        for p in (ypeer, xsib, zsib):
            pl.semaphore_signal(
                barrier_sem, inc=1,
                device_id=p, device_id_type=pl.DeviceIdType.MESH,
            )
        pl.semaphore_wait(barrier_sem, 3)

        def rdma(src_row, dst_row, nrows, ssem, rsem, peer):
            return pltpu.make_async_remote_copy(
                src_ref=out_ref.at[pl.ds(src_row, nrows), :],
                dst_ref=out_ref.at[pl.ds(dst_row, nrows), :],
                send_sem=ssem,
                recv_sem=rsem,
                device_id=peer,
                device_id_type=pl.DeviceIdType.MESH,
            )

        y_rd = {}
        for c in Y_ORDER:
            r = pltpu.make_async_remote_copy(
                src_ref=x_ref.at[
                    pl.ds(qoff + c * ck, ck),
                    pl.ds((1 - my_y) * n_out, n_out),
                ],
                dst_ref=out_ref.at[pl.ds(my_y * m + qoff + c * ck, ck), :],
                send_sem=ysend.at[c],
                recv_sem=yrecv.at[c],
                device_id=ypeer,
                device_id_type=pl.DeviceIdType.MESH,
            )
            r.start()
            y_rd[c] = r

        xq_rd, zq_rd = {}, {}
        for c in Y_ORDER:
            y_rd[c].wait_recv()
            r = rdma(row_q + c * ck, row_q + c * ck, ck,
                     xqs.at[c], xqr.at[c], xsib)
            r.start()
            xq_rd[c] = r
            r = rdma(row_q + c * ck, row_q + c * ck, ck,
                     zqs.at[c], zqr.at[c], zsib)
            r.start()
            zq_rd[c] = r

        xr_rd, zr_rd = {}, {}
        for i in range(NC // 2):
            lo, hi = i, NC - 1 - i
            zq_rd[lo].wait_recv()
            xr_rd[lo] = rdma(row_zq + lo * ck, row_zq + lo * ck, ck,
                             xrs.at[lo], xrr.at[lo], xsib)
            xr_rd[lo].start()
            xq_rd[hi].wait_recv()
            zr_rd[hi] = rdma(row_xq + hi * ck, row_xq + hi * ck, ck,
                             zrs.at[hi], zrr.at[hi], zsib)
            zr_rd[hi].start()

        for c in XR:
            xq_rd[c].wait_recv()
        for c in ZR:
            zq_rd[c].wait_recv()
        for k in XR:
            xr_rd[k].wait_recv()
        for k in ZR:
            zr_rd[k].wait_recv()
        for c in range(NC):
            y_rd[c].wait_send()
            xq_rd[c].wait_send()
            zq_rd[c].wait_send()
        for k in XR:
            xr_rd[k].wait_send()
        for k in ZR:
            zr_rd[k].wait_send()
        local_copy.wait()

    return pl.pallas_call(
        body,
        out_shape=jax.ShapeDtypeStruct((2 * m, n_out), x.dtype),
        in_specs=[pl.BlockSpec(memory_space=pl.ANY)],
        out_specs=pl.BlockSpec(memory_space=pl.ANY),
        scratch_shapes=[
            pltpu.SemaphoreType.DMA,
            pltpu.SemaphoreType.DMA((NC,)),
            pltpu.SemaphoreType.DMA((NC,)),
            pltpu.SemaphoreType.DMA((NC,)),
            pltpu.SemaphoreType.DMA((NC,)),
            pltpu.SemaphoreType.DMA((NC,)),
            pltpu.SemaphoreType.DMA((NC,)),
            pltpu.SemaphoreType.DMA((NC,)),
            pltpu.SemaphoreType.DMA((NC,)),
            pltpu.SemaphoreType.DMA((NC,)),
            pltpu.SemaphoreType.DMA((NC,)),
        ],
        compiler_params=pltpu.CompilerParams(collective_id=0),
    )(x)


# device time: 32491 ns/iter; 1.0741x vs baseline; 1.0423x over previous
import jax
import jax.numpy as jnp
from jax import lax
from jax.experimental import pallas as pl
from jax.experimental.pallas import tpu as pltpu

NC = 8
DY = (0, 1)
DX = (2, 3, 4)
DZ = (5, 6, 7)
Y_ORDER = (2, 5, 3, 6, 4, 7, 0, 1)


def kernel(x):
    m, n = x.shape
    n_out = n // 2
    qm = m // 4
    ck = qm // NC

    def body(x_ref, out_ref, local_sem,
             ysend, yrecv, xqs, xqr, zqs, zqr, xrs, xrr, zrs, zrr):
        my_x = lax.axis_index("x")
        my_y = lax.axis_index("y")
        my_z = lax.axis_index("z")
        ypeer = (my_x, 1 - my_y, my_z)
        xsib = (1 - my_x, my_y, my_z)
        zsib = (my_x, my_y, 1 - my_z)

        qoff = my_x * (2 * qm) + my_z * qm
        doff = (1 - my_x) * (2 * qm) + (1 - my_z) * qm
        recvbase = (1 - my_y) * m
        row_q = recvbase + qoff
        row_xq = recvbase + (1 - my_x) * (2 * qm) + my_z * qm
        row_zq = recvbase + my_x * (2 * qm) + (1 - my_z) * qm

        local_copy = pltpu.make_async_copy(
            x_ref.at[:, pl.ds(my_y * n_out, n_out)],
            out_ref.at[pl.ds(my_y * m, m), :],
            local_sem,
        )
        local_copy.start()

        barrier_sem = pltpu.get_barrier_semaphore()
        for p in (ypeer, xsib, zsib):
            pl.semaphore_signal(
                barrier_sem, inc=1,
                device_id=p, device_id_type=pl.DeviceIdType.MESH,
            )
        pl.semaphore_wait(barrier_sem, 3)

        def y_send(b_row, dst_row, sem_idx):
            return pltpu.make_async_remote_copy(
                src_ref=x_ref.at[
                    pl.ds(b_row, ck), pl.ds((1 - my_y) * n_out, n_out)
                ],
                dst_ref=out_ref.at[pl.ds(dst_row, ck), :],
                send_sem=ysend.at[sem_idx],
                recv_sem=yrecv.at[sem_idx],
                device_id=ypeer,
                device_id_type=pl.DeviceIdType.MESH,
            )

        def exch(row, ssem, rsem, peer):
            return pltpu.make_async_remote_copy(
                src_ref=out_ref.at[pl.ds(row, ck), :],
                dst_ref=out_ref.at[pl.ds(row, ck), :],
                send_sem=ssem,
                recv_sem=rsem,
                device_id=peer,
                device_id_type=pl.DeviceIdType.MESH,
            )

        y_rd, yd_rd = {}, {}
        for c in Y_ORDER:
            y_rd[c] = y_send(qoff + c * ck, my_y * m + qoff + c * ck, c)
            y_rd[c].start()
        for i, c in enumerate(DY):
            yd_rd[c] = y_send(doff + c * ck, my_y * m + doff + c * ck, NC + i)
            yd_rd[c].start()

        xq_rd, zq_rd = {}, {}
        for c in Y_ORDER:
            y_rd[c].wait_recv()
            xq_rd[c] = exch(row_q + c * ck, xqs.at[c], xqr.at[c], xsib)
            xq_rd[c].start()
            zq_rd[c] = exch(row_q + c * ck, zqs.at[c], zqr.at[c], zsib)
            zq_rd[c].start()

        xr_rd, zr_rd = {}, {}
        for cx, cz in zip(DX, DZ):
            zq_rd[cx].wait_recv()
            xr_rd[cx] = exch(row_zq + cx * ck, xrs.at[cx], xrr.at[cx], xsib)
            xr_rd[cx].start()
            xq_rd[cz].wait_recv()
            zr_rd[cz] = exch(row_xq + cz * ck, zrs.at[cz], zrr.at[cz], zsib)
            zr_rd[cz].start()

        for c in DY:
            yd_rd[c].wait_recv()
        for c in Y_ORDER:
            if c not in DX:
                zq_rd[c].wait_recv()
            if c not in DZ:
                xq_rd[c].wait_recv()
        for c in DX:
            xr_rd[c].wait_recv()
        for c in DZ:
            zr_rd[c].wait_recv()
        for c in range(NC):
            y_rd[c].wait_send()
            xq_rd[c].wait_send()
            zq_rd[c].wait_send()
        for c in DY:
            yd_rd[c].wait_send()
        for c in DX:
            xr_rd[c].wait_send()
        for c in DZ:
            zr_rd[c].wait_send()
        local_copy.wait()

    return pl.pallas_call(
        body,
        out_shape=jax.ShapeDtypeStruct((2 * m, n_out), x.dtype),
        in_specs=[pl.BlockSpec(memory_space=pl.ANY)],
        out_specs=pl.BlockSpec(memory_space=pl.ANY),
        scratch_shapes=[
            pltpu.SemaphoreType.DMA,
            pltpu.SemaphoreType.DMA((NC + len(DY),)),
            pltpu.SemaphoreType.DMA((NC + len(DY),)),
            pltpu.SemaphoreType.DMA((NC,)),
            pltpu.SemaphoreType.DMA((NC,)),
            pltpu.SemaphoreType.DMA((NC,)),
            pltpu.SemaphoreType.DMA((NC,)),
            pltpu.SemaphoreType.DMA((NC,)),
            pltpu.SemaphoreType.DMA((NC,)),
            pltpu.SemaphoreType.DMA((NC,)),
            pltpu.SemaphoreType.DMA((NC,)),
        ],
        compiler_params=pltpu.CompilerParams(collective_id=0),
    )(x)
